# Initial kernel scaffold; baseline (speedup 1.0000x reference)
import jax
import jax.numpy as jnp
from jax.experimental import pallas as pl


def kernel(pred, Laplace_W, Laplace_L, image_spiex):
    raise NotImplementedError("write your pallas kernel here")


# speedup vs baseline: 793.6855x; 793.6855x over previous
import functools

import jax
import jax.numpy as jnp
from jax import lax
from jax.experimental import pallas as pl
from jax.experimental.pallas import tpu as pltpu
from jax.experimental.pallas import tpu_sc as plsc

jax.config.update("jax_enable_x64", True)


def _z():
    return jnp.int32(0)

B = 4
C = 21
H = 512
WPX = 512
S = 1024
HW = H * WPX
ROWS_PER_IMG = HW // 128
NTILES = 32
TILES_PER_IMG = NTILES // B
ROWS_PER_TILE = ROWS_PER_IMG // TILES_PER_IMG
PIX_ROWS_PER_BLK = 64


def _stage1_body(pred_ref, lab_ref, psum_ref, laboff_ref):
    k = pl.program_id(0)
    psum_ref[...] = jnp.sum(pred_ref[...], axis=1)
    laboff_ref[...] = lab_ref[...] + k * jnp.int32(S)


def _stage1(pred, lab):
    nblk = H // PIX_ROWS_PER_BLK
    return pl.pallas_call(
        _stage1_body,
        grid=(B, nblk),
        in_specs=[
            pl.BlockSpec((1, C, PIX_ROWS_PER_BLK, WPX), lambda k, j: (k, _z(), j, _z())),
            pl.BlockSpec((1, ROWS_PER_IMG // nblk, 128), lambda k, j: (k, j, _z())),
        ],
        out_specs=[
            pl.BlockSpec((1, PIX_ROWS_PER_BLK, WPX), lambda k, j: (k, j, _z())),
            pl.BlockSpec((1, ROWS_PER_IMG // nblk, 128), lambda k, j: (k, j, _z())),
        ],
        out_shape=[
            jax.ShapeDtypeStruct((B, H, WPX), jnp.float32),
            jax.ShapeDtypeStruct((B, ROWS_PER_IMG, 128), jnp.int32),
        ],
    )(pred, lab)


_mesh = plsc.VectorSubcoreMesh(core_axis_name="c", subcore_axis_name="s")


PIX_PER_TILE = HW * B // NTILES


@functools.partial(
    pl.kernel,
    out_type=jax.ShapeDtypeStruct((2, 2, B * S), jnp.float32),
    mesh=_mesh,
    scratch_types=[
        pltpu.VMEM((PIX_PER_TILE,), jnp.float32),
        pltpu.VMEM((PIX_PER_TILE,), jnp.int32),
        pltpu.VMEM((PIX_PER_TILE,), jnp.float32),
        pltpu.VMEM((B * S,), jnp.float32),
        pltpu.VMEM_SHARED((B * S,), jnp.float32),
        pltpu.VMEM_SHARED((B * S,), jnp.float32),
    ],
)
def _segsum(psum_hbm, lab_hbm, out_hbm, val_v, idx_v, ones_v, zero_v, acc_s, acc_c):
    cid = lax.axis_index("c")
    sid = lax.axis_index("s")
    wid = cid * jnp.int32(16) + sid
    img = wid // jnp.int32(TILES_PER_IMG)
    chunk = wid % jnp.int32(TILES_PER_IMG)

    def fill_ones(i, carry):
        ones_v[pl.ds(i * jnp.int32(16), 16)] = jnp.full((16,), 1.0, jnp.float32)
        return carry

    lax.fori_loop(jnp.int32(0), jnp.int32(PIX_PER_TILE // 16), fill_ones, 0)

    @pl.when(sid == 0)
    def _():
        def fz(i, carry):
            zero_v[pl.ds(i * jnp.int32(16), 16)] = jnp.zeros((16,), jnp.float32)
            return carry

        lax.fori_loop(jnp.int32(0), jnp.int32((B * S) // 16), fz, 0)
        pltpu.sync_copy(zero_v, acc_s)
        pltpu.sync_copy(zero_v, acc_c)

    pltpu.sync_copy(psum_hbm.at[img, pl.ds(chunk * jnp.int32(PIX_PER_TILE), PIX_PER_TILE)], val_v)
    pltpu.sync_copy(lab_hbm.at[img, pl.ds(chunk * jnp.int32(PIX_PER_TILE), PIX_PER_TILE)], idx_v)

    plsc.subcore_barrier()

    pltpu.sync_copy(val_v, acc_s.at[idx_v], add=True)
    pltpu.sync_copy(ones_v, acc_c.at[idx_v], add=True)

    plsc.subcore_barrier()

    @pl.when(sid == 0)
    def _():
        pltpu.sync_copy(acc_s, out_hbm.at[cid, jnp.int32(0)])
        pltpu.sync_copy(acc_c, out_hbm.at[cid, jnp.int32(1)])


def _stage3_body(sc_ref, l_ref, w_ref, out_ref, acc_ref):
    k = pl.program_id(0)
    sums = sc_ref[0, 0, 0] + sc_ref[0, 1, 0]
    cnts = sc_ref[0, 0, 1] + sc_ref[0, 1, 1]
    vals = jnp.where(cnts > 0.0, sums / jnp.maximum(cnts, 1.0), 0.0)
    vals = vals.reshape(1, S)
    y = jnp.dot(vals, l_ref[0], preferred_element_type=jnp.float32)
    acc_ref[k, 0] = jnp.sum(y * vals)
    acc_ref[k, 1] = jnp.sum(w_ref[0] * w_ref[0])

    @pl.when(k == B - 1)
    def _():
        tot = 0.0
        for i in range(B):
            tot += (2.0 * C) / jnp.sqrt(acc_ref[i, 1]) * acc_ref[i, 0]
        out_ref[0, 0] = tot


def _stage3(sc_out, lap_l, lap_w):
    return pl.pallas_call(
        _stage3_body,
        grid=(B,),
        in_specs=[
            pl.BlockSpec((1, 2, 2, S), lambda k: (k, _z(), _z(), _z())),
            pl.BlockSpec((1, S, S), lambda k: (k, _z(), _z())),
            pl.BlockSpec((1, S, S), lambda k: (k, _z(), _z())),
        ],
        out_specs=pl.BlockSpec((1, 1), lambda k: (_z(), _z()), memory_space=pltpu.SMEM),
        out_shape=jax.ShapeDtypeStruct((1, 1), jnp.float32),
        scratch_shapes=[pltpu.SMEM((B, 2), jnp.float32)],
        compiler_params=pltpu.CompilerParams(dimension_semantics=("arbitrary",)),
    )(sc_out, lap_l, lap_w)


def kernel(pred, Laplace_W, Laplace_L, image_spiex):
    lab = image_spiex.astype(jnp.int32).reshape(B, ROWS_PER_IMG, 128)
    psum, laboff = _stage1(pred, lab)
    sc_out = _segsum(psum.reshape(B, HW), laboff.reshape(B, HW))
    sc4 = jnp.transpose(sc_out.reshape(2, 2, B, S), (2, 0, 1, 3))
    loss = _stage3(sc4, Laplace_L, Laplace_W)
    return loss[0, 0].astype(jnp.float64)


# speedup vs baseline: 1066.2887x; 1.3435x over previous
import functools

import jax
import jax.numpy as jnp
from jax import lax
from jax.experimental import pallas as pl
from jax.experimental.pallas import tpu as pltpu
from jax.experimental.pallas import tpu_sc as plsc

jax.config.update("jax_enable_x64", True)


def _z():
    return jnp.int32(0)


B = 4
C = 21
H = 512
WPX = 512
S = 1024
HW = H * WPX
NTILES = 32
TILES_PER_IMG = NTILES // B
PIX_PER_TILE = HW // TILES_PER_IMG
PIX_ROWS_PER_BLK = 64


def _stage1_body(pred_ref, psum_ref):
    psum_ref[...] = jnp.sum(pred_ref[...], axis=1)


def _stage1(pred):
    nblk = H // PIX_ROWS_PER_BLK
    return pl.pallas_call(
        _stage1_body,
        grid=(B, nblk),
        in_specs=[
            pl.BlockSpec((1, C, PIX_ROWS_PER_BLK, WPX), lambda k, j: (k, _z(), j, _z())),
        ],
        out_specs=pl.BlockSpec((1, PIX_ROWS_PER_BLK, WPX), lambda k, j: (k, j, _z())),
        out_shape=jax.ShapeDtypeStruct((B, H, WPX), jnp.float32),
    )(pred)


_mesh = plsc.VectorSubcoreMesh(core_axis_name="c", subcore_axis_name="s")


@functools.partial(
    pl.kernel,
    out_type=jax.ShapeDtypeStruct((B, 2, 2, S), jnp.float32),
    mesh=_mesh,
    scratch_types=[
        pltpu.VMEM((PIX_PER_TILE,), jnp.float32),
        pltpu.VMEM((PIX_PER_TILE,), jnp.int32),
        pltpu.VMEM((PIX_PER_TILE,), jnp.float32),
        pltpu.VMEM((B * S,), jnp.float32),
        pltpu.VMEM_SHARED((B * S,), jnp.float32),
        pltpu.VMEM_SHARED((B * S,), jnp.float32),
    ],
)
def _segsum(psum_hbm, lab_hbm, out_hbm, val_v, idx_v, ones_v, zero_v, acc_s, acc_c):
    cid = lax.axis_index("c")
    sid = lax.axis_index("s")
    wid = cid * jnp.int32(16) + sid
    img = wid // jnp.int32(TILES_PER_IMG)
    chunk = wid % jnp.int32(TILES_PER_IMG)

    def fill_ones(i, carry):
        ones_v[pl.ds(i * jnp.int32(16), 16)] = jnp.full((16,), 1.0, jnp.float32)
        return carry

    lax.fori_loop(jnp.int32(0), jnp.int32(PIX_PER_TILE // 16), fill_ones, 0)

    @pl.when(sid == 0)
    def _():
        def fz(i, carry):
            zero_v[pl.ds(i * jnp.int32(16), 16)] = jnp.zeros((16,), jnp.float32)
            return carry

        lax.fori_loop(jnp.int32(0), jnp.int32((B * S) // 16), fz, 0)
        pltpu.sync_copy(zero_v, acc_s)
        pltpu.sync_copy(zero_v, acc_c)

    pltpu.sync_copy(psum_hbm.at[img, pl.ds(chunk * jnp.int32(PIX_PER_TILE), PIX_PER_TILE)], val_v)
    pltpu.sync_copy(lab_hbm.at[img, pl.ds(chunk * jnp.int32(PIX_PER_TILE), PIX_PER_TILE)], idx_v)

    plsc.subcore_barrier()

    base = img * jnp.int32(S)
    pltpu.sync_copy(val_v, acc_s.at[pl.ds(base, S)].at[idx_v], add=True)
    pltpu.sync_copy(ones_v, acc_c.at[pl.ds(base, S)].at[idx_v], add=True)

    plsc.subcore_barrier()

    @pl.when(sid == 0)
    def _():
        for i in range(B):
            pltpu.sync_copy(acc_s.at[pl.ds(jnp.int32(i * S), S)],
                            out_hbm.at[jnp.int32(i), cid, jnp.int32(0)])
            pltpu.sync_copy(acc_c.at[pl.ds(jnp.int32(i * S), S)],
                            out_hbm.at[jnp.int32(i), cid, jnp.int32(1)])


def _stage3_body(sc_ref, l_ref, w_ref, out_ref, acc_ref):
    k = pl.program_id(0)
    sums = sc_ref[0, 0, 0] + sc_ref[0, 1, 0]
    cnts = sc_ref[0, 0, 1] + sc_ref[0, 1, 1]
    vals = jnp.where(cnts > 0.0, sums / jnp.maximum(cnts, 1.0), 0.0)
    vals = vals.reshape(1, S)
    y = jnp.dot(vals, l_ref[0], preferred_element_type=jnp.float32)
    acc_ref[k, 0] = jnp.sum(y * vals)
    acc_ref[k, 1] = jnp.sum(w_ref[0] * w_ref[0])

    @pl.when(k == B - 1)
    def _():
        tot = 0.0
        for i in range(B):
            tot += (2.0 * C) / jnp.sqrt(acc_ref[i, 1]) * acc_ref[i, 0]
        out_ref[0, 0] = tot


def _stage3(sc_out, lap_l, lap_w):
    return pl.pallas_call(
        _stage3_body,
        grid=(B,),
        in_specs=[
            pl.BlockSpec((1, 2, 2, S), lambda k: (k, _z(), _z(), _z())),
            pl.BlockSpec((1, S, S), lambda k: (k, _z(), _z())),
            pl.BlockSpec((1, S, S), lambda k: (k, _z(), _z())),
        ],
        out_specs=pl.BlockSpec((1, 1), lambda k: (_z(), _z()), memory_space=pltpu.SMEM),
        out_shape=jax.ShapeDtypeStruct((1, 1), jnp.float32),
        scratch_shapes=[pltpu.SMEM((B, 2), jnp.float32)],
        compiler_params=pltpu.CompilerParams(dimension_semantics=("arbitrary",)),
    )(sc_out, lap_l, lap_w)


def kernel(pred, Laplace_W, Laplace_L, image_spiex):
    lab = image_spiex.astype(jnp.int32).reshape(B, HW)
    psum = _stage1(pred)
    sc_out = _segsum(psum.reshape(B, HW), lab)
    loss = _stage3(sc_out, Laplace_L, Laplace_W)
    return loss[0, 0].astype(jnp.float64)


# speedup vs baseline: 1120.8173x; 1.0511x over previous
import functools

import jax
import jax.numpy as jnp
from jax import lax
from jax.experimental import pallas as pl
from jax.experimental.pallas import tpu as pltpu
from jax.experimental.pallas import tpu_sc as plsc

jax.config.update("jax_enable_x64", True)


def _z():
    return jnp.int32(0)


B = 4
C = 21
H = 512
WPX = 512
S = 1024
HW = H * WPX
NTILES = 32
TILES_PER_IMG = NTILES // B
PIX_PER_TILE = HW // TILES_PER_IMG
PIX_ROWS_PER_BLK = 64


def _stage1_body(pred_ref, psum_ref):
    psum_ref[...] = jnp.sum(pred_ref[...], axis=1).reshape(1, 1, PIX_PER_TILE)


def _stage1(pred):
    nblk = H // PIX_ROWS_PER_BLK
    return pl.pallas_call(
        _stage1_body,
        grid=(B, nblk),
        in_specs=[
            pl.BlockSpec((1, C, PIX_ROWS_PER_BLK, WPX), lambda k, j: (k, _z(), j, _z())),
        ],
        out_specs=pl.BlockSpec((1, 1, PIX_PER_TILE),
                               lambda k, j: (k * jnp.int32(8) + j, _z(), _z())),
        out_shape=jax.ShapeDtypeStruct((NTILES, 1, PIX_PER_TILE), jnp.float32),
    )(pred)


_mesh = plsc.VectorSubcoreMesh(core_axis_name="c", subcore_axis_name="s")


@functools.partial(
    pl.kernel,
    out_type=jax.ShapeDtypeStruct((B, 2, 2, S), jnp.float32),
    mesh=_mesh,
    scratch_types=[
        pltpu.VMEM((PIX_PER_TILE,), jnp.float32),
        pltpu.VMEM((PIX_PER_TILE,), jnp.int32),
        pltpu.VMEM((PIX_PER_TILE,), jnp.float32),
        pltpu.VMEM((B * S,), jnp.float32),
        pltpu.VMEM_SHARED((B * S,), jnp.float32),
        pltpu.VMEM_SHARED((B * S,), jnp.float32),
    ],
)
def _segsum(psum_hbm, lab_hbm, out_hbm, val_v, idx_v, ones_v, zero_v, acc_s, acc_c):
    cid = lax.axis_index("c")
    sid = lax.axis_index("s")
    wid = cid * jnp.int32(16) + sid
    img = wid // jnp.int32(TILES_PER_IMG)
    chunk = wid % jnp.int32(TILES_PER_IMG)

    def fill_ones(i, carry):
        ones_v[pl.ds(i * jnp.int32(16), 16)] = jnp.full((16,), 1.0, jnp.float32)
        return carry

    lax.fori_loop(jnp.int32(0), jnp.int32(PIX_PER_TILE // 16), fill_ones, 0)

    @pl.when(sid == 0)
    def _():
        def fz(i, carry):
            zero_v[pl.ds(i * jnp.int32(16), 16)] = jnp.zeros((16,), jnp.float32)
            return carry

        lax.fori_loop(jnp.int32(0), jnp.int32((B * S) // 16), fz, 0)
        pltpu.sync_copy(zero_v, acc_s)
        pltpu.sync_copy(zero_v, acc_c)

    pltpu.sync_copy(psum_hbm.at[wid, jnp.int32(0)], val_v)
    pltpu.sync_copy(lab_hbm.at[img, pl.ds(chunk * jnp.int32(PIX_PER_TILE), PIX_PER_TILE)], idx_v)

    plsc.subcore_barrier()

    base = img * jnp.int32(S)
    pltpu.sync_copy(val_v, acc_s.at[pl.ds(base, S)].at[idx_v], add=True)
    pltpu.sync_copy(ones_v, acc_c.at[pl.ds(base, S)].at[idx_v], add=True)

    plsc.subcore_barrier()

    @pl.when(sid == 0)
    def _():
        for i in range(B):
            pltpu.sync_copy(acc_s.at[pl.ds(jnp.int32(i * S), S)],
                            out_hbm.at[jnp.int32(i), cid, jnp.int32(0)])
            pltpu.sync_copy(acc_c.at[pl.ds(jnp.int32(i * S), S)],
                            out_hbm.at[jnp.int32(i), cid, jnp.int32(1)])


def _stage3_body(sc_ref, l_ref, w_ref, out_ref, acc_ref):
    k = pl.program_id(0)
    sums = sc_ref[0, 0, 0] + sc_ref[0, 1, 0]
    cnts = sc_ref[0, 0, 1] + sc_ref[0, 1, 1]
    vals = jnp.where(cnts > 0.0, sums / jnp.maximum(cnts, 1.0), 0.0)
    vals = vals.reshape(1, S)
    y = jnp.dot(vals, l_ref[0], preferred_element_type=jnp.float32)
    acc_ref[k, 0] = jnp.sum(y * vals)
    acc_ref[k, 1] = jnp.sum(w_ref[0] * w_ref[0])

    @pl.when(k == B - 1)
    def _():
        tot = 0.0
        for i in range(B):
            tot += (2.0 * C) / jnp.sqrt(acc_ref[i, 1]) * acc_ref[i, 0]
        out_ref[0, 0] = tot


def _stage3(sc_out, lap_l, lap_w):
    return pl.pallas_call(
        _stage3_body,
        grid=(B,),
        in_specs=[
            pl.BlockSpec((1, 2, 2, S), lambda k: (k, _z(), _z(), _z())),
            pl.BlockSpec((1, S, S), lambda k: (k, _z(), _z())),
            pl.BlockSpec((1, S, S), lambda k: (k, _z(), _z())),
        ],
        out_specs=pl.BlockSpec((1, 1), lambda k: (_z(), _z()), memory_space=pltpu.SMEM),
        out_shape=jax.ShapeDtypeStruct((1, 1), jnp.float32),
        scratch_shapes=[pltpu.SMEM((B, 2), jnp.float32)],
        compiler_params=pltpu.CompilerParams(dimension_semantics=("arbitrary",)),
    )(sc_out, lap_l, lap_w)


def kernel(pred, Laplace_W, Laplace_L, image_spiex):
    lab = image_spiex.astype(jnp.int32).reshape(B, HW)
    psum = _stage1(pred)
    sc_out = _segsum(psum, lab)
    loss = _stage3(sc_out, Laplace_L, Laplace_W)
    return loss[0, 0].astype(jnp.float64)


# speedup vs baseline: 1158.6795x; 1.0338x over previous
import functools

import jax
import jax.numpy as jnp
from jax import lax
from jax.experimental import pallas as pl
from jax.experimental.pallas import tpu as pltpu
from jax.experimental.pallas import tpu_sc as plsc

jax.config.update("jax_enable_x64", True)


def _z():
    return jnp.int32(0)


B = 4
C = 21
H = 512
WPX = 512
S = 1024
HW = H * WPX
NTILES = 32
TILES_PER_IMG = NTILES // B
PIX_PER_TILE = HW // TILES_PER_IMG
PIX_ROWS_PER_BLK = 64


def _stage1_body(pred_ref, psum_ref):
    psum_ref[...] = jnp.sum(pred_ref[...], axis=1).reshape(1, 1, PIX_PER_TILE)


def _stage1(pred):
    nblk = H // PIX_ROWS_PER_BLK
    return pl.pallas_call(
        _stage1_body,
        grid=(B, nblk),
        in_specs=[
            pl.BlockSpec((1, C, PIX_ROWS_PER_BLK, WPX), lambda k, j: (k, _z(), j, _z())),
        ],
        out_specs=pl.BlockSpec((1, 1, PIX_PER_TILE),
                               lambda k, j: (k * jnp.int32(8) + j, _z(), _z())),
        out_shape=jax.ShapeDtypeStruct((NTILES, 1, PIX_PER_TILE), jnp.float32),
    )(pred)


_mesh = plsc.VectorSubcoreMesh(core_axis_name="c", subcore_axis_name="s")


@functools.partial(
    pl.kernel,
    out_type=jax.ShapeDtypeStruct((B, 2, 2, S), jnp.float32),
    mesh=_mesh,
    scratch_types=[
        pltpu.VMEM((PIX_PER_TILE,), jnp.float32),
        pltpu.VMEM((PIX_PER_TILE,), jnp.int32),
        pltpu.VMEM((PIX_PER_TILE,), jnp.float32),
        pltpu.VMEM((B * S,), jnp.float32),
        pltpu.VMEM_SHARED((B * S,), jnp.float32),
        pltpu.VMEM_SHARED((B * S,), jnp.float32),
        pltpu.SemaphoreType.DMA,
        pltpu.SemaphoreType.DMA,
        pltpu.SemaphoreType.DMA,
        pltpu.SemaphoreType.DMA,
    ],
)
def _segsum(psum_hbm, lab_hbm, out_hbm, val_v, idx_v, ones_v, zero_v, acc_s, acc_c,
            sem1, sem2, sem3, sem4):
    cid = lax.axis_index("c")
    sid = lax.axis_index("s")
    wid = cid * jnp.int32(16) + sid
    img = wid // jnp.int32(TILES_PER_IMG)
    chunk = wid % jnp.int32(TILES_PER_IMG)

    d1 = pltpu.async_copy(psum_hbm.at[wid, jnp.int32(0)], val_v, sem1)
    d2 = pltpu.async_copy(
        lab_hbm.at[img, pl.ds(chunk * jnp.int32(PIX_PER_TILE), PIX_PER_TILE)], idx_v, sem2)

    def fill_ones(i, carry):
        ones_v[pl.ds(i * jnp.int32(16), 16)] = jnp.full((16,), 1.0, jnp.float32)
        return carry

    lax.fori_loop(jnp.int32(0), jnp.int32(PIX_PER_TILE // 16), fill_ones, 0)

    @pl.when(sid == 0)
    def _():
        def fz(i, carry):
            zero_v[pl.ds(i * jnp.int32(16), 16)] = jnp.zeros((16,), jnp.float32)
            return carry

        lax.fori_loop(jnp.int32(0), jnp.int32((B * S) // 16), fz, 0)
        pltpu.sync_copy(zero_v, acc_s)
        pltpu.sync_copy(zero_v, acc_c)

    d1.wait()
    d2.wait()

    plsc.subcore_barrier()

    base = img * jnp.int32(S)
    s1 = pltpu.async_copy(val_v, acc_s.at[pl.ds(base, S)].at[idx_v], sem3, add=True)
    s2 = pltpu.async_copy(ones_v, acc_c.at[pl.ds(base, S)].at[idx_v], sem4, add=True)
    s1.wait()
    s2.wait()

    plsc.subcore_barrier()

    @pl.when(sid == 0)
    def _():
        for i in range(B):
            pltpu.sync_copy(acc_s.at[pl.ds(jnp.int32(i * S), S)],
                            out_hbm.at[jnp.int32(i), cid, jnp.int32(0)])
            pltpu.sync_copy(acc_c.at[pl.ds(jnp.int32(i * S), S)],
                            out_hbm.at[jnp.int32(i), cid, jnp.int32(1)])


def _stage3_body(sc_ref, l_ref, w_ref, out_ref, acc_ref):
    k = pl.program_id(0)
    sums = sc_ref[0, 0, 0] + sc_ref[0, 1, 0]
    cnts = sc_ref[0, 0, 1] + sc_ref[0, 1, 1]
    vals = jnp.where(cnts > 0.0, sums / jnp.maximum(cnts, 1.0), 0.0)
    vals = vals.reshape(1, S)
    y = jnp.dot(vals, l_ref[0], preferred_element_type=jnp.float32)
    acc_ref[k, 0] = jnp.sum(y * vals)
    acc_ref[k, 1] = jnp.sum(w_ref[0] * w_ref[0])

    @pl.when(k == B - 1)
    def _():
        tot = 0.0
        for i in range(B):
            tot += (2.0 * C) / jnp.sqrt(acc_ref[i, 1]) * acc_ref[i, 0]
        out_ref[0, 0] = tot


def _stage3(sc_out, lap_l, lap_w):
    return pl.pallas_call(
        _stage3_body,
        grid=(B,),
        in_specs=[
            pl.BlockSpec((1, 2, 2, S), lambda k: (k, _z(), _z(), _z())),
            pl.BlockSpec((1, S, S), lambda k: (k, _z(), _z())),
            pl.BlockSpec((1, S, S), lambda k: (k, _z(), _z())),
        ],
        out_specs=pl.BlockSpec((1, 1), lambda k: (_z(), _z()), memory_space=pltpu.SMEM),
        out_shape=jax.ShapeDtypeStruct((1, 1), jnp.float32),
        scratch_shapes=[pltpu.SMEM((B, 2), jnp.float32)],
        compiler_params=pltpu.CompilerParams(dimension_semantics=("arbitrary",)),
    )(sc_out, lap_l, lap_w)


def kernel(pred, Laplace_W, Laplace_L, image_spiex):
    lab = image_spiex.astype(jnp.int32).reshape(B, HW)
    psum = _stage1(pred)
    sc_out = _segsum(psum, lab)
    loss = _stage3(sc_out, Laplace_L, Laplace_W)
    return loss[0, 0].astype(jnp.float64)


# speedup vs baseline: 1210.2624x; 1.0445x over previous
import functools

import jax
import jax.numpy as jnp
from jax import lax
from jax.experimental import pallas as pl
from jax.experimental.pallas import tpu as pltpu
from jax.experimental.pallas import tpu_sc as plsc

jax.config.update("jax_enable_x64", True)


def _z():
    return jnp.int32(0)


B = 4
C = 21
H = 512
WPX = 512
S = 1024
HW = H * WPX
NTILES = 32
TILES_PER_IMG = NTILES // B
PIX_PER_TILE = HW // TILES_PER_IMG
PIX_ROWS_PER_BLK = 64


def _stage1_body(pred_ref, psum_ref):
    psum_ref[...] = jnp.sum(pred_ref[...], axis=1).reshape(1, 1, PIX_PER_TILE)


def _stage1(pred):
    nblk = H // PIX_ROWS_PER_BLK
    return pl.pallas_call(
        _stage1_body,
        grid=(B, nblk),
        in_specs=[
            pl.BlockSpec((1, C, PIX_ROWS_PER_BLK, WPX), lambda k, j: (k, _z(), j, _z())),
        ],
        out_specs=pl.BlockSpec((1, 1, PIX_PER_TILE),
                               lambda k, j: (k * jnp.int32(8) + j, _z(), _z())),
        out_shape=jax.ShapeDtypeStruct((NTILES, 1, PIX_PER_TILE), jnp.float32),
    )(pred)


_mesh = plsc.VectorSubcoreMesh(core_axis_name="c", subcore_axis_name="s")


@functools.partial(
    pl.kernel,
    out_type=jax.ShapeDtypeStruct((B, 2, 2, S), jnp.float32),
    mesh=_mesh,
    scratch_types=[
        pltpu.VMEM((PIX_PER_TILE,), jnp.float32),
        pltpu.VMEM((PIX_PER_TILE,), jnp.int32),
        pltpu.VMEM((PIX_PER_TILE,), jnp.float32),
        pltpu.VMEM((B * S,), jnp.float32),
        pltpu.VMEM_SHARED((B * S,), jnp.float32),
        pltpu.VMEM_SHARED((B * S,), jnp.float32),
        pltpu.SemaphoreType.DMA,
        pltpu.SemaphoreType.DMA,
        pltpu.SemaphoreType.DMA,
        pltpu.SemaphoreType.DMA,
    ],
)
def _segsum(psum_hbm, lab_hbm, out_hbm, val_v, idx_v, ones_v, zero_v, acc_s, acc_c,
            sem1, sem2, sem3, sem4):
    cid = lax.axis_index("c")
    sid = lax.axis_index("s")
    wid = cid * jnp.int32(16) + sid
    img = wid // jnp.int32(TILES_PER_IMG)
    chunk = wid % jnp.int32(TILES_PER_IMG)

    d1 = pltpu.async_copy(psum_hbm.at[wid, jnp.int32(0)], val_v, sem1)
    d2 = pltpu.async_copy(
        lab_hbm.at[img, pl.ds(chunk * jnp.int32(PIX_PER_TILE), PIX_PER_TILE)], idx_v, sem2)

    def fill_ones(i, carry):
        ones_v[pl.ds(i * jnp.int32(16), 16)] = jnp.full((16,), 1.0, jnp.float32)
        return carry

    lax.fori_loop(jnp.int32(0), jnp.int32(PIX_PER_TILE // 16), fill_ones, 0)

    @pl.when(sid == 0)
    def _():
        def fz(i, carry):
            zero_v[pl.ds(i * jnp.int32(16), 16)] = jnp.zeros((16,), jnp.float32)
            return carry

        lax.fori_loop(jnp.int32(0), jnp.int32((B * S) // 16), fz, 0)
        pltpu.sync_copy(zero_v, acc_s)
        pltpu.sync_copy(zero_v, acc_c)

    d1.wait()
    d2.wait()

    plsc.subcore_barrier()

    base = img * jnp.int32(S)
    s1 = pltpu.async_copy(val_v, acc_s.at[pl.ds(base, S)].at[idx_v], sem3, add=True)
    s2 = pltpu.async_copy(ones_v, acc_c.at[pl.ds(base, S)].at[idx_v], sem4, add=True)
    s1.wait()
    s2.wait()

    plsc.subcore_barrier()

    @pl.when(sid == 0)
    def _():
        for i in range(B):
            pltpu.sync_copy(acc_s.at[pl.ds(jnp.int32(i * S), S)],
                            out_hbm.at[jnp.int32(i), cid, jnp.int32(0)])
            pltpu.sync_copy(acc_c.at[pl.ds(jnp.int32(i * S), S)],
                            out_hbm.at[jnp.int32(i), cid, jnp.int32(1)])


def _stage3a_body(w_ref, out_ref):
    k = pl.program_id(0)
    out_ref[k, 0] = jnp.sum(w_ref[0] * w_ref[0])


def _stage3a(lap_w):
    return pl.pallas_call(
        _stage3a_body,
        grid=(B,),
        in_specs=[pl.BlockSpec((1, S, S), lambda k: (k, _z(), _z()))],
        out_specs=pl.BlockSpec((B, 1), lambda k: (_z(), _z()), memory_space=pltpu.SMEM),
        out_shape=jax.ShapeDtypeStruct((B, 1), jnp.float32),
        compiler_params=pltpu.CompilerParams(dimension_semantics=("arbitrary",)),
    )(lap_w)


def _stage3b_body(sc_ref, l_ref, w2_ref, out_ref, acc_ref):
    k = pl.program_id(0)
    sums = sc_ref[0, 0, 0] + sc_ref[0, 1, 0]
    cnts = sc_ref[0, 0, 1] + sc_ref[0, 1, 1]
    vals = jnp.where(cnts > 0.0, sums / jnp.maximum(cnts, 1.0), 0.0)
    vals = vals.reshape(1, S)
    y = jnp.dot(vals, l_ref[0], preferred_element_type=jnp.float32)
    acc_ref[k, 0] = jnp.sum(y * vals)

    @pl.when(k == B - 1)
    def _():
        tot = 0.0
        for i in range(B):
            tot += (2.0 * C) / jnp.sqrt(w2_ref[i, 0]) * acc_ref[i, 0]
        out_ref[0, 0] = tot


def _stage3b(sc_out, lap_l, w2):
    return pl.pallas_call(
        _stage3b_body,
        grid=(B,),
        in_specs=[
            pl.BlockSpec((1, 2, 2, S), lambda k: (k, _z(), _z(), _z())),
            pl.BlockSpec((1, S, S), lambda k: (k, _z(), _z())),
            pl.BlockSpec((B, 1), lambda k: (_z(), _z()), memory_space=pltpu.SMEM),
        ],
        out_specs=pl.BlockSpec((1, 1), lambda k: (_z(), _z()), memory_space=pltpu.SMEM),
        out_shape=jax.ShapeDtypeStruct((1, 1), jnp.float32),
        scratch_shapes=[pltpu.SMEM((B, 1), jnp.float32)],
        compiler_params=pltpu.CompilerParams(dimension_semantics=("arbitrary",)),
    )(sc_out, lap_l, w2)


def kernel(pred, Laplace_W, Laplace_L, image_spiex):
    lab = image_spiex.astype(jnp.int32).reshape(B, HW)
    psum = _stage1(pred)
    w2 = _stage3a(Laplace_W)
    sc_out = _segsum(psum, lab)
    loss = _stage3b(sc_out, Laplace_L, w2)
    return loss[0, 0].astype(jnp.float64)
